# baseline (speedup 1.0000x reference)
import functools

import jax
import jax.numpy as jnp
from jax import lax
from jax.experimental import pallas as pl
from jax.experimental.pallas import tpu as pltpu
from jax.experimental.pallas import tpu_sc as plsc

_B = 16384
_D = 16
_PKW = 128

_NC = 2
_NS = 16
_NW = _NC * _NS
_BPW = _B // _NW
_CH = 256
_NCHUNK = _BPW // _CH


def _sc_gather(upk, ipk, cpk, uhi, ihi, chi):
    mesh = plsc.VectorSubcoreMesh(core_axis_name="c", subcore_axis_name="s")

    @functools.partial(
        pl.kernel,
        mesh=mesh,
        out_type=(
            jax.ShapeDtypeStruct((_B, _PKW), jnp.int32),
            jax.ShapeDtypeStruct((_B, _PKW), jnp.int32),
            jax.ShapeDtypeStruct((_B, _PKW), jnp.int32),
        ),
        scratch_types=[
            pltpu.VMEM((_BPW,), jnp.int32),
            pltpu.VMEM((_BPW,), jnp.int32),
            pltpu.VMEM((_BPW,), jnp.int32),
            pltpu.VMEM((_CH, _PKW), jnp.int32),
            pltpu.VMEM((_CH, _PKW), jnp.int32),
            pltpu.VMEM((_CH, _PKW), jnp.int32),
            pltpu.SemaphoreType.DMA,
            pltpu.SemaphoreType.DMA,
            pltpu.SemaphoreType.DMA,
        ],
    )
    def k(ut, it, ct, ui, ii, ci, uo, io, co,
          uiv, iiv, civ, urv, irv, crv, s0, s1, s2):
        uw = ut.bitcast(jnp.int32)
        iw = it.bitcast(jnp.int32)
        cw = ct.bitcast(jnp.int32)
        wid = lax.axis_index("s") * _NC + lax.axis_index("c")
        base = wid * _BPW
        sl = pl.ds(base, _BPW)
        pltpu.sync_copy(ui.at[sl], uiv)
        pltpu.sync_copy(ii.at[sl], iiv)
        pltpu.sync_copy(ci.at[sl], civ)

        @pl.loop(0, _NCHUNK)
        def _(c):
            off = c * _CH
            isl = pl.ds(off, _CH)
            osl = pl.ds(base + off, _CH)
            c0 = pltpu.async_copy(uw.at[uiv.at[isl]], urv, s0)
            c1 = pltpu.async_copy(iw.at[iiv.at[isl]], irv, s1)
            c2 = pltpu.async_copy(cw.at[civ.at[isl]], crv, s2)
            c0.wait()
            c1.wait()
            c2.wait()
            pltpu.sync_copy(urv, uo.at[osl])
            pltpu.sync_copy(irv, io.at[osl])
            pltpu.sync_copy(crv, co.at[osl])

    return k(upk, ipk, cpk, uhi, ihi, chi)


_BLK = 2048


def _dense_body(pu_ref, pi_ref, pc_ref, uid_ref, iid_ref, cid_ref,
                aid_ref, gid_ref, p_ref, agetab_ref, gentab_ref,
                uwe_ref, uwo_ref, uw1b_ref, uw1c_ref, ub1_ref, uw2_ref, ub2_ref,
                iwe_ref, iwo_ref, cwe_ref, cwo_ref, iw1c_ref, ib1_ref,
                iw2_ref, ib2_ref, o_ref):
    dot = lambda a, b: jnp.dot(a, b, preferred_element_type=jnp.float32)
    n = pu_ref.shape[0]
    embl = lax.broadcasted_iota(jnp.int32, (n, _PKW), 1) >> 4

    def tower_in(p_ref, id_ref, we_ref, wo_ref):
        w = p_ref[...]
        low = lax.bitcast_convert_type(w << 16, jnp.float32)
        high = lax.bitcast_convert_type(w & jnp.int32(-65536), jnp.float32)
        lo = id_ref[...] & 15
        m = jnp.where(embl == (lo & 7), jnp.where(lo < 8, low, high), 0.0)
        return dot(m, we_ref[...])

    cu = tower_in(pu_ref, uid_ref, uwe_ref, uwo_ref)
    ci = tower_in(pi_ref, iid_ref, iwe_ref, iwo_ref)
    cc = tower_in(pc_ref, cid_ref, cwe_ref, cwo_ref)

    oh_a = (aid_ref[...] == lax.broadcasted_iota(jnp.int32, (n, 128), 1))
    ae = dot(oh_a.astype(jnp.float32), agetab_ref[...])
    oh_g = (gid_ref[...] == lax.broadcasted_iota(jnp.int32, (n, 8), 1))
    ge = dot(oh_g.astype(jnp.float32), gentab_ref[...])

    h = jnp.maximum(
        cu + dot(ae, uw1b_ref[...]) + dot(ge, uw1c_ref[...]) + ub1_ref[...],
        0.0)
    uv = dot(h, uw2_ref[...]) + ub2_ref[...]

    h2 = jnp.maximum(
        ci + cc + p_ref[...] * (1.0 / 1000.0) * iw1c_ref[...] + ib1_ref[...],
        0.0)
    iv = dot(h2, iw2_ref[...]) + ib2_ref[...]

    num = jnp.sum(uv * iv, axis=1, keepdims=True)
    du = jnp.maximum(jnp.sqrt(jnp.sum(uv * uv, axis=1, keepdims=True)), 1e-12)
    di = jnp.maximum(jnp.sqrt(jnp.sum(iv * iv, axis=1, keepdims=True)), 1e-12)
    o_ref[...] = num / (du * di)


def _dense(pu, pi, pc, uid2, iid2, cid2, aid, gid, price, agetab, gentab,
           uwe, uwo, uW1b, uW1c, ub1, uW2, ub2,
           iwe, iwo, cwe, cwo, iW1c, ib1, iW2, ib2):
    full = lambda shape: pl.BlockSpec(shape, lambda i: (0, 0))
    rowpk = lambda: pl.BlockSpec((_BLK, _PKW), lambda i: (i, 0))
    col1 = lambda: pl.BlockSpec((_BLK, 1), lambda i: (i, 0))
    return pl.pallas_call(
        _dense_body,
        grid=(_B // _BLK,),
        in_specs=[
            rowpk(), rowpk(), rowpk(),
            col1(), col1(), col1(), col1(), col1(), col1(),
            full((128, _D)),
            full((8, _D)),
            full((_PKW, _D)),
            full((_PKW, _D)),
            full((_D, _D)),
            full((_D, _D)),
            full((1, _D)),
            full((_D, _D)),
            full((1, _D)),
            full((_PKW, _D)),
            full((_PKW, _D)),
            full((_PKW, _D)),
            full((_PKW, _D)),
            full((1, _D)),
            full((1, _D)),
            full((_D, _D)),
            full((1, _D)),
        ],
        out_specs=pl.BlockSpec((_BLK, 1), lambda i: (i, 0)),
        out_shape=jax.ShapeDtypeStruct((_B, 1), jnp.float32),
    )(pu, pi, pc, uid2, iid2, cid2, aid, gid, price, agetab, gentab,
      uwe, uwo, uW1b, uW1c, ub1, uW2, ub2,
      iwe, iwo, cwe, cwo, iW1c, ib1, iW2, ib2)


def _rep16(w16):
    w = jnp.tile(w16, (8, 1))
    return w, w


def kernel(user_id, age, gender, item_id, category, price,
           user_table, age_table, gender_table, item_table, cat_table,
           uW1, ub1, uW2, ub2, iW1, ib1, iW2, ib2):
    uid = user_id[:, 0].astype(jnp.int32)
    iid = item_id[:, 0].astype(jnp.int32)
    cid = category[:, 0].astype(jnp.int32)

    upk = user_table.astype(jnp.bfloat16).reshape(-1, 128)
    ipk = item_table.astype(jnp.bfloat16).reshape(-1, 128)
    cpk = jnp.pad(cat_table, ((0, 24), (0, 8))).astype(jnp.bfloat16).reshape(-1, 128)

    pu, pi, pc = _sc_gather(upk, ipk, cpk, uid >> 4, iid >> 4, cid >> 4)

    agetab = jnp.pad(age_table, ((0, 28), (0, 8)))
    gentab = jnp.pad(gender_table, ((0, 5), (0, 12)))

    z8 = jnp.zeros((8, _D), jnp.float32)
    z12 = jnp.zeros((12, _D), jnp.float32)
    uwe, uwo = _rep16(uW1[0:16])
    iwe, iwo = _rep16(iW1[0:16])
    cwe, cwo = _rep16(jnp.concatenate([iW1[16:24], z8], axis=0))
    uW1b = jnp.concatenate([uW1[16:24], z8], axis=0)
    uW1c = jnp.concatenate([uW1[24:28], z12], axis=0)
    iW1c = iW1[24:25]

    logit = _dense(pu, pi, pc,
                   user_id.astype(jnp.int32), item_id.astype(jnp.int32),
                   category.astype(jnp.int32),
                   age.astype(jnp.int32), gender.astype(jnp.int32), price,
                   agetab, gentab,
                   uwe, uwo, uW1b, uW1c, ub1.reshape(1, _D), uW2, ub2.reshape(1, _D),
                   iwe, iwo, cwe, cwo, iW1c, ib1.reshape(1, _D), iW2, ib2.reshape(1, _D))
    return logit[:, 0]


# speedup vs baseline: 1.0152x; 1.0152x over previous
import functools

import jax
import jax.numpy as jnp
from jax import lax
from jax.experimental import pallas as pl
from jax.experimental.pallas import tpu as pltpu
from jax.experimental.pallas import tpu_sc as plsc

_B = 16384
_D = 16
_PKW = 128

_NC = 2
_NS = 16
_NW = _NC * _NS
_BPW = _B // _NW
_CH = 256
_NCHUNK = _BPW // _CH


def _sc_gather(upk, ipk, uhi, ihi):
    mesh = plsc.VectorSubcoreMesh(core_axis_name="c", subcore_axis_name="s")

    @functools.partial(
        pl.kernel,
        mesh=mesh,
        out_type=(
            jax.ShapeDtypeStruct((_B, _PKW), jnp.int32),
            jax.ShapeDtypeStruct((_B, _PKW), jnp.int32),
        ),
        scratch_types=[
            pltpu.VMEM((_BPW,), jnp.int32),
            pltpu.VMEM((_BPW,), jnp.int32),
            pltpu.VMEM((_CH, _PKW), jnp.int32),
            pltpu.VMEM((_CH, _PKW), jnp.int32),
            pltpu.SemaphoreType.DMA,
            pltpu.SemaphoreType.DMA,
        ],
    )
    def k(ut, it, ui, ii, uo, io,
          uiv, iiv, urv, irv, s0, s1):
        uw = ut.bitcast(jnp.int32)
        iw = it.bitcast(jnp.int32)
        wid = lax.axis_index("s") * _NC + lax.axis_index("c")
        base = wid * _BPW
        sl = pl.ds(base, _BPW)
        pltpu.sync_copy(ui.at[sl], uiv)
        pltpu.sync_copy(ii.at[sl], iiv)

        @pl.loop(0, _NCHUNK)
        def _(c):
            off = c * _CH
            isl = pl.ds(off, _CH)
            osl = pl.ds(base + off, _CH)
            c0 = pltpu.async_copy(uw.at[uiv.at[isl]], urv, s0)
            c1 = pltpu.async_copy(iw.at[iiv.at[isl]], irv, s1)
            c0.wait()
            c1.wait()
            pltpu.sync_copy(urv, uo.at[osl])
            pltpu.sync_copy(irv, io.at[osl])

    return k(upk, ipk, uhi, ihi)


_BLK = 2048


def _dense_body(pu_ref, pi_ref, uid_ref, iid_ref, cid_ref,
                aid_ref, gid_ref, p_ref, agetab_ref, gentab_ref, cattab_ref,
                uwe_ref, uwo_ref, uw1b_ref, uw1c_ref, ub1_ref, uw2_ref, ub2_ref,
                iwe_ref, iwo_ref, cw1b_ref, iw1c_ref, ib1_ref,
                iw2_ref, ib2_ref, o_ref):
    dot = lambda a, b: jnp.dot(a, b, preferred_element_type=jnp.float32)
    n = pu_ref.shape[0]
    embl = lax.broadcasted_iota(jnp.int32, (n, _PKW), 1) >> 4

    def tower_in(p_ref, id_ref, we_ref, wo_ref):
        w = p_ref[...]
        low = lax.bitcast_convert_type(w << 16, jnp.float32)
        high = lax.bitcast_convert_type(w & jnp.int32(-65536), jnp.float32)
        lo = id_ref[...] & 15
        m = jnp.where(embl == (lo & 7), jnp.where(lo < 8, low, high), 0.0)
        return dot(m, we_ref[...])

    cu = tower_in(pu_ref, uid_ref, uwe_ref, uwo_ref)
    ci = tower_in(pi_ref, iid_ref, iwe_ref, iwo_ref)

    catw = dot(cattab_ref[...], cw1b_ref[...])
    oh_c = (cid_ref[...] == lax.broadcasted_iota(jnp.int32, (n, 1024), 1))
    cc = dot(oh_c.astype(jnp.float32), catw)

    oh_a = (aid_ref[...] == lax.broadcasted_iota(jnp.int32, (n, 128), 1))
    ae = dot(oh_a.astype(jnp.float32), agetab_ref[...])
    oh_g = (gid_ref[...] == lax.broadcasted_iota(jnp.int32, (n, 8), 1))
    ge = dot(oh_g.astype(jnp.float32), gentab_ref[...])

    h = jnp.maximum(
        cu + dot(ae, uw1b_ref[...]) + dot(ge, uw1c_ref[...]) + ub1_ref[...],
        0.0)
    uv = dot(h, uw2_ref[...]) + ub2_ref[...]

    h2 = jnp.maximum(
        ci + cc + p_ref[...] * (1.0 / 1000.0) * iw1c_ref[...] + ib1_ref[...],
        0.0)
    iv = dot(h2, iw2_ref[...]) + ib2_ref[...]

    num = jnp.sum(uv * iv, axis=1, keepdims=True)
    du = jnp.maximum(jnp.sqrt(jnp.sum(uv * uv, axis=1, keepdims=True)), 1e-12)
    di = jnp.maximum(jnp.sqrt(jnp.sum(iv * iv, axis=1, keepdims=True)), 1e-12)
    o_ref[...] = num / (du * di)


def _dense(pu, pi, uid2, iid2, cid2, aid, gid, price, agetab, gentab, cattab,
           uwe, uwo, uW1b, uW1c, ub1, uW2, ub2,
           iwe, iwo, cW1b, iW1c, ib1, iW2, ib2):
    full = lambda shape: pl.BlockSpec(shape, lambda i: (0, 0))
    rowpk = lambda: pl.BlockSpec((_BLK, _PKW), lambda i: (i, 0))
    col1 = lambda: pl.BlockSpec((_BLK, 1), lambda i: (i, 0))
    return pl.pallas_call(
        _dense_body,
        grid=(_B // _BLK,),
        in_specs=[
            rowpk(), rowpk(),
            col1(), col1(), col1(), col1(), col1(), col1(),
            full((128, _D)),
            full((8, _D)),
            full((1024, _D)),
            full((_PKW, _D)),
            full((_PKW, _D)),
            full((_D, _D)),
            full((_D, _D)),
            full((1, _D)),
            full((_D, _D)),
            full((1, _D)),
            full((_PKW, _D)),
            full((_PKW, _D)),
            full((_D, _D)),
            full((1, _D)),
            full((1, _D)),
            full((_D, _D)),
            full((1, _D)),
        ],
        out_specs=pl.BlockSpec((_BLK, 1), lambda i: (i, 0)),
        out_shape=jax.ShapeDtypeStruct((_B, 1), jnp.float32),
    )(pu, pi, uid2, iid2, cid2, aid, gid, price, agetab, gentab, cattab,
      uwe, uwo, uW1b, uW1c, ub1, uW2, ub2,
      iwe, iwo, cW1b, iW1c, ib1, iW2, ib2)


def _rep16(w16):
    w = jnp.tile(w16, (8, 1))
    return w, w


def kernel(user_id, age, gender, item_id, category, price,
           user_table, age_table, gender_table, item_table, cat_table,
           uW1, ub1, uW2, ub2, iW1, ib1, iW2, ib2):
    uid = user_id[:, 0].astype(jnp.int32)
    iid = item_id[:, 0].astype(jnp.int32)
    cid = category[:, 0].astype(jnp.int32)

    upk = user_table.astype(jnp.bfloat16).reshape(-1, 128)
    ipk = item_table.astype(jnp.bfloat16).reshape(-1, 128)

    pu, pi = _sc_gather(upk, ipk, uid >> 4, iid >> 4)

    agetab = jnp.pad(age_table, ((0, 28), (0, 8)))
    gentab = jnp.pad(gender_table, ((0, 5), (0, 12)))
    cattab = jnp.pad(cat_table, ((0, 24), (0, 8)))

    z8 = jnp.zeros((8, _D), jnp.float32)
    z12 = jnp.zeros((12, _D), jnp.float32)
    uwe, uwo = _rep16(uW1[0:16])
    iwe, iwo = _rep16(iW1[0:16])
    uW1b = jnp.concatenate([uW1[16:24], z8], axis=0)
    uW1c = jnp.concatenate([uW1[24:28], z12], axis=0)
    cW1b = jnp.concatenate([iW1[16:24], z8], axis=0)
    iW1c = iW1[24:25]

    logit = _dense(pu, pi,
                   user_id.astype(jnp.int32), item_id.astype(jnp.int32),
                   category.astype(jnp.int32),
                   age.astype(jnp.int32), gender.astype(jnp.int32), price,
                   agetab, gentab, cattab,
                   uwe, uwo, uW1b, uW1c, ub1.reshape(1, _D), uW2, ub2.reshape(1, _D),
                   iwe, iwo, cW1b, iW1c, ib1.reshape(1, _D), iW2, ib2.reshape(1, _D))
    return logit[:, 0]


# speedup vs baseline: 1.0158x; 1.0006x over previous
import functools

import jax
import jax.numpy as jnp
from jax import lax
from jax.experimental import pallas as pl
from jax.experimental.pallas import tpu as pltpu
from jax.experimental.pallas import tpu_sc as plsc

_B = 16384
_D = 16
_PKW = 128

_NC = 2
_NS = 16
_NW = _NC * _NS
_BPW = _B // _NW
_CH = 256
_NCHUNK = _BPW // _CH


def _sc_gather(upk, ipk, uhi, ihi):
    mesh = plsc.VectorSubcoreMesh(core_axis_name="c", subcore_axis_name="s")

    @functools.partial(
        pl.kernel,
        mesh=mesh,
        out_type=(
            jax.ShapeDtypeStruct((_B, _PKW), jnp.int32),
            jax.ShapeDtypeStruct((_B, _PKW), jnp.int32),
        ),
        scratch_types=[
            pltpu.VMEM((_BPW,), jnp.int32),
            pltpu.VMEM((_BPW,), jnp.int32),
            pltpu.VMEM((_CH, _PKW), jnp.int32),
            pltpu.VMEM((_CH, _PKW), jnp.int32),
            pltpu.SemaphoreType.DMA,
            pltpu.SemaphoreType.DMA,
        ],
    )
    def k(ut, it, ui, ii, uo, io,
          uiv, iiv, urv, irv, s0, s1):
        uw = ut.bitcast(jnp.int32)
        iw = it.bitcast(jnp.int32)
        wid = lax.axis_index("s") * _NC + lax.axis_index("c")
        base = wid * _BPW
        sl = pl.ds(base, _BPW)
        pltpu.sync_copy(ui.at[sl], uiv)
        pltpu.sync_copy(ii.at[sl], iiv)

        @pl.loop(0, _NCHUNK)
        def _(c):
            off = c * _CH
            isl = pl.ds(off, _CH)
            osl = pl.ds(base + off, _CH)
            c0 = pltpu.async_copy(uw.at[uiv.at[isl]], urv, s0)
            c1 = pltpu.async_copy(iw.at[iiv.at[isl]], irv, s1)
            c0.wait()
            c1.wait()
            pltpu.sync_copy(urv, uo.at[osl])
            pltpu.sync_copy(irv, io.at[osl])

    return k(upk, ipk, uhi, ihi)


_BLK = 2048


def _dense_body(pu_ref, pi_ref, uid_ref, iid_ref, cid_ref,
                aid_ref, gid_ref, p_ref, agetab_ref, gentab_ref, cattab_ref,
                uwe_ref, uw1b_ref, uw1c_ref, ub1_ref, uw2_ref, ub2_ref,
                iwe_ref, cw1b_ref, iw1c_ref, ib1_ref,
                iw2_ref, ib2_ref, o_ref):
    dot = lambda a, b: jnp.dot(a, b, preferred_element_type=jnp.float32)
    n = pu_ref.shape[0]
    embl = lax.broadcasted_iota(jnp.int32, (n, _PKW), 1) >> 4

    def tower_in(p_ref, id_ref, we_ref):
        w = p_ref[...]
        low = lax.bitcast_convert_type(w << 16, jnp.float32)
        high = lax.bitcast_convert_type(w & jnp.int32(-65536), jnp.float32)
        lo = id_ref[...] & 15
        m = jnp.where(embl == (lo & 7), jnp.where(lo < 8, low, high), 0.0)
        return dot(m, we_ref[...])

    cu = tower_in(pu_ref, uid_ref, uwe_ref)
    ci = tower_in(pi_ref, iid_ref, iwe_ref)

    catw = dot(cattab_ref[...], cw1b_ref[...])
    oh_c = (cid_ref[...] == lax.broadcasted_iota(jnp.int32, (n, 1024), 1))
    cc = dot(oh_c.astype(jnp.float32), catw)

    oh_a = (aid_ref[...] == lax.broadcasted_iota(jnp.int32, (n, 128), 1))
    ae = dot(oh_a.astype(jnp.float32), agetab_ref[...])
    oh_g = (gid_ref[...] == lax.broadcasted_iota(jnp.int32, (n, 8), 1))
    ge = dot(oh_g.astype(jnp.float32), gentab_ref[...])

    h = jnp.maximum(
        cu + dot(ae, uw1b_ref[...]) + dot(ge, uw1c_ref[...]) + ub1_ref[...],
        0.0)
    uv = dot(h, uw2_ref[...]) + ub2_ref[...]

    h2 = jnp.maximum(
        ci + cc + p_ref[...] * (1.0 / 1000.0) * iw1c_ref[...] + ib1_ref[...],
        0.0)
    iv = dot(h2, iw2_ref[...]) + ib2_ref[...]

    num = jnp.sum(uv * iv, axis=1, keepdims=True)
    du = jnp.maximum(jnp.sqrt(jnp.sum(uv * uv, axis=1, keepdims=True)), 1e-12)
    di = jnp.maximum(jnp.sqrt(jnp.sum(iv * iv, axis=1, keepdims=True)), 1e-12)
    o_ref[...] = num / (du * di)


def _dense(pu, pi, uid2, iid2, cid2, aid, gid, price, agetab, gentab, cattab,
           uwe, uW1b, uW1c, ub1, uW2, ub2,
           iwe, cW1b, iW1c, ib1, iW2, ib2):
    full = lambda shape: pl.BlockSpec(shape, lambda i: (0, 0))
    rowpk = lambda: pl.BlockSpec((_BLK, _PKW), lambda i: (i, 0))
    col1 = lambda: pl.BlockSpec((_BLK, 1), lambda i: (i, 0))
    return pl.pallas_call(
        _dense_body,
        grid=(_B // _BLK,),
        in_specs=[
            rowpk(), rowpk(),
            col1(), col1(), col1(), col1(), col1(), col1(),
            full((128, _D)),
            full((8, _D)),
            full((1024, _D)),
            full((_PKW, _D)),
            full((_D, _D)),
            full((_D, _D)),
            full((1, _D)),
            full((_D, _D)),
            full((1, _D)),
            full((_PKW, _D)),
            full((_D, _D)),
            full((1, _D)),
            full((1, _D)),
            full((_D, _D)),
            full((1, _D)),
        ],
        out_specs=pl.BlockSpec((_BLK, 1), lambda i: (i, 0)),
        out_shape=jax.ShapeDtypeStruct((_B, 1), jnp.float32),
    )(pu, pi, uid2, iid2, cid2, aid, gid, price, agetab, gentab, cattab,
      uwe, uW1b, uW1c, ub1, uW2, ub2,
      iwe, cW1b, iW1c, ib1, iW2, ib2)


def _rep8(w16):
    return jnp.tile(w16, (8, 1))


def kernel(user_id, age, gender, item_id, category, price,
           user_table, age_table, gender_table, item_table, cat_table,
           uW1, ub1, uW2, ub2, iW1, ib1, iW2, ib2):
    uid = user_id[:, 0].astype(jnp.int32)
    iid = item_id[:, 0].astype(jnp.int32)
    cid = category[:, 0].astype(jnp.int32)

    upk = user_table.astype(jnp.bfloat16).reshape(-1, 128)
    ipk = item_table.astype(jnp.bfloat16).reshape(-1, 128)

    pu, pi = _sc_gather(upk, ipk, uid >> 4, iid >> 4)

    agetab = jnp.pad(age_table, ((0, 28), (0, 8)))
    gentab = jnp.pad(gender_table, ((0, 5), (0, 12)))
    cattab = jnp.pad(cat_table, ((0, 24), (0, 8)))

    z8 = jnp.zeros((8, _D), jnp.float32)
    z12 = jnp.zeros((12, _D), jnp.float32)
    uwe = _rep8(uW1[0:16])
    iwe = _rep8(iW1[0:16])
    uW1b = jnp.concatenate([uW1[16:24], z8], axis=0)
    uW1c = jnp.concatenate([uW1[24:28], z12], axis=0)
    cW1b = jnp.concatenate([iW1[16:24], z8], axis=0)
    iW1c = iW1[24:25]

    logit = _dense(pu, pi,
                   user_id.astype(jnp.int32), item_id.astype(jnp.int32),
                   category.astype(jnp.int32),
                   age.astype(jnp.int32), gender.astype(jnp.int32), price,
                   agetab, gentab, cattab,
                   uwe, uW1b, uW1c, ub1.reshape(1, _D), uW2, ub2.reshape(1, _D),
                   iwe, cW1b, iW1c, ib1.reshape(1, _D), iW2, ib2.reshape(1, _D))
    return logit[:, 0]
